# baseline (speedup 1.0000x reference)
import functools

import jax
import jax.numpy as jnp
import numpy as np
from jax.experimental import pallas as pl
from jax.experimental.pallas import tpu as pltpu

_K = 20
_RB = 256
_KB = 512
_MB = 512


def _consts(n):
    noise = jax.random.uniform(
        jax.random.key(42), (n, n), dtype=jnp.float32) * 0.01
    tri = jnp.triu(jnp.ones((n, n), jnp.bfloat16))
    return noise, tri


def _adj_topk_kernel(nv1_ref, nv2_ref, noise_ref, tri_ref, wm_ref, deg_ref):
    i = pl.program_id(0)
    rb, n = noise_ref.shape
    a1 = jnp.tanh(2.0 * nv1_ref[...])
    a2 = jnp.tanh(2.0 * nv2_ref[...])
    logits = jax.lax.dot_general(a1, a2, (((1,), (1,)), ((), ())),
                                 preferred_element_type=jnp.float32)
    adj = jax.nn.relu(jnp.tanh(2.0 * logits))
    v = adj + noise_ref[...]

    vb = jax.lax.bitcast_convert_type(v, jnp.int32)
    kf = jnp.float32(_K)

    def vbody(_, carry):
        lo, hi = carry
        mid = lo + (hi - lo) // 2
        cnt = jnp.sum(jnp.where(vb >= mid, 1.0, 0.0), axis=1, keepdims=True)
        ok = cnt >= kf
        return jnp.where(ok, mid, lo), jnp.where(ok, hi, mid)

    lo0 = jnp.zeros((rb, 1), jnp.int32)
    hi0 = jnp.full((rb, 1), jnp.int32(1 << 30))
    th, _ = jax.lax.fori_loop(0, 30, vbody, (lo0, hi0))

    gt = vb > th
    eq = vb == th
    n_gt = jnp.sum(jnp.where(gt, 1.0, 0.0), axis=1, keepdims=True)
    need = kf - n_gt

    eq_bf = jnp.where(eq, 1.0, 0.0).astype(jnp.bfloat16)
    cum = jax.lax.dot_general(eq_bf, tri_ref[...], (((1,), (0,)), ((), ())),
                              preferred_element_type=jnp.float32)
    msk = gt | (eq & (cum <= need))

    wm = jnp.where(msk, adj, 0.0)
    rows = jax.lax.broadcasted_iota(jnp.int32, (rb, n), 0) + i * rb
    cols = jax.lax.broadcasted_iota(jnp.int32, (rb, n), 1)
    dmask = rows == cols
    diag_adj = jnp.sum(jnp.where(dmask, adj, 0.0), axis=1)
    diag_sel = jnp.sum(jnp.where(dmask & msk, 1.0, 0.0), axis=1)
    has_self = (diag_sel > 0.0) & (diag_adj > 0.0)
    loop_w = jnp.where(has_self, 0.0, 1.0)
    wm = wm + jnp.where(dmask, loop_w[:, None], 0.0)
    wm_ref[...] = wm.astype(jnp.bfloat16)
    part = jnp.sum(wm, axis=0, keepdims=True)

    @pl.when(i == 0)
    def _():
        deg_ref[...] = part

    @pl.when(i > 0)
    def _():
        deg_ref[...] = deg_ref[...] + part


def _xw_kernel(x_ref, cw_ref, deg_ref, xw_ref):
    xb = x_ref[...].astype(jnp.bfloat16)
    xw = jax.lax.dot_general(xb, cw_ref[...], (((2,), (0,)), ((), ())),
                             preferred_element_type=jnp.float32)
    deg_i = deg_ref[...]
    dinv_i = jnp.where(deg_i > 0.0, jax.lax.rsqrt(deg_i), 0.0)
    xw_ref[...] = (xw * dinv_i[0][None, :, None]).astype(jnp.bfloat16)


def _agg_kernel(wm_ref, xw_ref, deg_ref, bft_ref, y_ref, acc_ref):
    mb = pl.program_id(0)
    ib = pl.program_id(1)
    nib = pl.num_programs(1)
    bb = xw_ref.shape[0]
    contrib = jax.lax.dot_general(
        wm_ref[...], xw_ref[...], (((0,), (1,)), ((), ())),
        preferred_element_type=jnp.float32)

    @pl.when(ib == 0)
    def _():
        acc_ref[...] = contrib

    @pl.when(ib > 0)
    def _():
        acc_ref[...] = acc_ref[...] + contrib

    @pl.when(ib == nib - 1)
    def _():
        deg_m = deg_ref[0, pl.ds(mb * _MB, _MB)]
        dinv_m = jnp.where(deg_m > 0.0, jax.lax.rsqrt(deg_m), 0.0)
        y_ref[...] = jnp.maximum(
            acc_ref[...] * dinv_m[:, None, None] + bft_ref[...][None], 0.0)


def kernel(x, nodevec1, nodevec2, W, b):
    Bb, Nn, Fin, Tt = x.shape
    Fout = W.shape[1]
    FT = Fout * Tt
    noise, tri = _consts(Nn)

    wm, deg = pl.pallas_call(
        _adj_topk_kernel,
        grid=(Nn // _RB,),
        in_specs=[
            pl.BlockSpec((_RB, nodevec1.shape[1]), lambda i: (i, 0)),
            pl.BlockSpec((Nn, nodevec2.shape[1]), lambda i: (0, 0)),
            pl.BlockSpec((_RB, Nn), lambda i: (i, 0)),
            pl.BlockSpec((Nn, Nn), lambda i: (0, 0)),
        ],
        out_specs=[
            pl.BlockSpec((_RB, Nn), lambda i: (i, 0)),
            pl.BlockSpec((1, Nn), lambda i: (0, 0)),
        ],
        out_shape=[
            jax.ShapeDtypeStruct((Nn, Nn), jnp.bfloat16),
            jax.ShapeDtypeStruct((1, Nn), jnp.float32),
        ],
    )(nodevec1, nodevec2, noise, tri)

    cw = jnp.kron(W, jnp.eye(Tt, dtype=W.dtype)).astype(jnp.bfloat16)
    xflat = x.reshape(Bb, Nn, Fin * Tt)
    xw = pl.pallas_call(
        _xw_kernel,
        grid=(Nn // _RB,),
        in_specs=[
            pl.BlockSpec((Bb, _RB, Fin * Tt), lambda i: (0, i, 0)),
            pl.BlockSpec((Fin * Tt, FT), lambda i: (0, 0)),
            pl.BlockSpec((1, _RB), lambda i: (0, i)),
        ],
        out_specs=pl.BlockSpec((Bb, _RB, FT), lambda i: (0, i, 0)),
        out_shape=jax.ShapeDtypeStruct((Bb, Nn, FT), jnp.bfloat16),
    )(xflat, cw, deg)

    bft = jnp.repeat(b, Tt)[None, :]
    y = pl.pallas_call(
        _agg_kernel,
        grid=(Nn // _MB, Nn // _KB),
        in_specs=[
            pl.BlockSpec((_KB, _MB), lambda m, i: (i, m)),
            pl.BlockSpec((Bb, _KB, FT), lambda m, i: (0, i, 0)),
            pl.BlockSpec((1, Nn), lambda m, i: (0, 0)),
            pl.BlockSpec((1, FT), lambda m, i: (0, 0)),
        ],
        out_specs=pl.BlockSpec((_MB, Bb, FT), lambda m, i: (m, 0, 0)),
        out_shape=jax.ShapeDtypeStruct((Nn, Bb, FT), jnp.float32),
        scratch_shapes=[pltpu.VMEM((_MB, Bb, FT), jnp.float32)],
    )(wm, xw, deg, bft)

    return jnp.transpose(y.reshape(Nn, Bb, Fout, Tt), (1, 0, 2, 3))


# speedup vs baseline: 1.1906x; 1.1906x over previous
import functools

import jax
import jax.numpy as jnp
import numpy as np
from jax.experimental import pallas as pl
from jax.experimental.pallas import tpu as pltpu

_K = 20
_RB = 256
_KB = 512
_MB = 512


@functools.lru_cache(maxsize=2)
def _consts(n):
    with jax.ensure_compile_time_eval():
        noise = np.asarray(
            jax.random.uniform(jax.random.key(42), (n, n), dtype=jnp.float32)
            * 0.01)
        tri = np.asarray(jnp.triu(jnp.ones((n, n), jnp.bfloat16)))
    return noise, tri


def _adj_topk_kernel(nv1_ref, nv2_ref, noise_ref, tri_ref, wm_ref, deg_ref):
    i = pl.program_id(0)
    rb, n = noise_ref.shape
    a1 = jnp.tanh(2.0 * nv1_ref[...])
    a2 = jnp.tanh(2.0 * nv2_ref[...])
    logits = jax.lax.dot_general(a1, a2, (((1,), (1,)), ((), ())),
                                 preferred_element_type=jnp.float32)
    adj = jax.nn.relu(jnp.tanh(2.0 * logits))
    v = adj + noise_ref[...]

    vb = jax.lax.bitcast_convert_type(v, jnp.int32)
    kf = jnp.float32(_K)

    def vbody(_, carry):
        lo, hi = carry
        mid = lo + (hi - lo) // 2
        cnt = jnp.sum(jnp.where(vb >= mid, 1.0, 0.0), axis=1, keepdims=True)
        ok = cnt >= kf
        return jnp.where(ok, mid, lo), jnp.where(ok, hi, mid)

    lo0 = jnp.zeros((rb, 1), jnp.int32)
    hi0 = jnp.full((rb, 1), jnp.int32(1 << 30))
    th, _ = jax.lax.fori_loop(0, 30, vbody, (lo0, hi0))

    gt = vb > th
    eq = vb == th
    n_gt = jnp.sum(jnp.where(gt, 1.0, 0.0), axis=1, keepdims=True)
    need = kf - n_gt

    eq_bf = jnp.where(eq, 1.0, 0.0).astype(jnp.bfloat16)
    cum = jax.lax.dot_general(eq_bf, tri_ref[...], (((1,), (0,)), ((), ())),
                              preferred_element_type=jnp.float32)
    msk = gt | (eq & (cum <= need))

    wm = jnp.where(msk, adj, 0.0)
    rows = jax.lax.broadcasted_iota(jnp.int32, (rb, n), 0) + i * rb
    cols = jax.lax.broadcasted_iota(jnp.int32, (rb, n), 1)
    dmask = rows == cols
    diag_adj = jnp.sum(jnp.where(dmask, adj, 0.0), axis=1)
    diag_sel = jnp.sum(jnp.where(dmask & msk, 1.0, 0.0), axis=1)
    has_self = (diag_sel > 0.0) & (diag_adj > 0.0)
    loop_w = jnp.where(has_self, 0.0, 1.0)
    wm = wm + jnp.where(dmask, loop_w[:, None], 0.0)
    wm_ref[...] = wm.astype(jnp.bfloat16)
    part = jnp.sum(wm, axis=0, keepdims=True)

    @pl.when(i == 0)
    def _():
        deg_ref[...] = part

    @pl.when(i > 0)
    def _():
        deg_ref[...] = deg_ref[...] + part


def _xw_kernel(x_ref, cw_ref, deg_ref, xw_ref):
    xb = x_ref[...].astype(jnp.bfloat16)
    xw = jax.lax.dot_general(xb, cw_ref[...], (((2,), (0,)), ((), ())),
                             preferred_element_type=jnp.float32)
    deg_i = deg_ref[...]
    dinv_i = jnp.where(deg_i > 0.0, jax.lax.rsqrt(deg_i), 0.0)
    xw_ref[...] = (xw * dinv_i[0][None, :, None]).astype(jnp.bfloat16)


def _agg_kernel(wm_ref, xw_ref, deg_ref, bft_ref, y_ref, acc_ref):
    mb = pl.program_id(0)
    ib = pl.program_id(1)
    nib = pl.num_programs(1)
    bb = xw_ref.shape[0]
    contrib = jax.lax.dot_general(
        wm_ref[...], xw_ref[...], (((0,), (1,)), ((), ())),
        preferred_element_type=jnp.float32)

    @pl.when(ib == 0)
    def _():
        acc_ref[...] = contrib

    @pl.when(ib > 0)
    def _():
        acc_ref[...] = acc_ref[...] + contrib

    @pl.when(ib == nib - 1)
    def _():
        deg_m = deg_ref[0, pl.ds(mb * _MB, _MB)]
        dinv_m = jnp.where(deg_m > 0.0, jax.lax.rsqrt(deg_m), 0.0)
        y_ref[...] = jnp.maximum(
            acc_ref[...] * dinv_m[:, None, None] + bft_ref[...][None], 0.0)


def kernel(x, nodevec1, nodevec2, W, b):
    Bb, Nn, Fin, Tt = x.shape
    Fout = W.shape[1]
    FT = Fout * Tt
    noise, tri = _consts(Nn)

    wm, deg = pl.pallas_call(
        _adj_topk_kernel,
        grid=(Nn // _RB,),
        in_specs=[
            pl.BlockSpec((_RB, nodevec1.shape[1]), lambda i: (i, 0)),
            pl.BlockSpec((Nn, nodevec2.shape[1]), lambda i: (0, 0)),
            pl.BlockSpec((_RB, Nn), lambda i: (i, 0)),
            pl.BlockSpec((Nn, Nn), lambda i: (0, 0)),
        ],
        out_specs=[
            pl.BlockSpec((_RB, Nn), lambda i: (i, 0)),
            pl.BlockSpec((1, Nn), lambda i: (0, 0)),
        ],
        out_shape=[
            jax.ShapeDtypeStruct((Nn, Nn), jnp.bfloat16),
            jax.ShapeDtypeStruct((1, Nn), jnp.float32),
        ],
    )(nodevec1, nodevec2, noise, tri)

    cw = jnp.kron(W, jnp.eye(Tt, dtype=W.dtype)).astype(jnp.bfloat16)
    xflat = x.reshape(Bb, Nn, Fin * Tt)
    xw = pl.pallas_call(
        _xw_kernel,
        grid=(Nn // _RB,),
        in_specs=[
            pl.BlockSpec((Bb, _RB, Fin * Tt), lambda i: (0, i, 0)),
            pl.BlockSpec((Fin * Tt, FT), lambda i: (0, 0)),
            pl.BlockSpec((1, _RB), lambda i: (0, i)),
        ],
        out_specs=pl.BlockSpec((Bb, _RB, FT), lambda i: (0, i, 0)),
        out_shape=jax.ShapeDtypeStruct((Bb, Nn, FT), jnp.bfloat16),
    )(xflat, cw, deg)

    bft = jnp.repeat(b, Tt)[None, :]
    y = pl.pallas_call(
        _agg_kernel,
        grid=(Nn // _MB, Nn // _KB),
        in_specs=[
            pl.BlockSpec((_KB, _MB), lambda m, i: (i, m)),
            pl.BlockSpec((Bb, _KB, FT), lambda m, i: (0, i, 0)),
            pl.BlockSpec((1, Nn), lambda m, i: (0, 0)),
            pl.BlockSpec((1, FT), lambda m, i: (0, 0)),
        ],
        out_specs=pl.BlockSpec((_MB, Bb, FT), lambda m, i: (m, 0, 0)),
        out_shape=jax.ShapeDtypeStruct((Nn, Bb, FT), jnp.float32),
        scratch_shapes=[pltpu.VMEM((_MB, Bb, FT), jnp.float32)],
    )(wm, xw, deg, bft)

    return jnp.transpose(y.reshape(Nn, Bb, Fout, Tt), (1, 0, 2, 3))
